# baseline (speedup 1.0000x reference)
import numpy as np
import jax
import jax.numpy as jnp
from jax import lax
from jax.experimental import pallas as pl
from jax.experimental.pallas import tpu as pltpu

B, C, H, W = 16, 256, 16, 16
K = 8192
MS = [1, 2, 3, 4, 5, 6, 8, 10, 13, 16]
NLEV = len(MS)
KT = 256
HW = H * W
TMAX = B * HW
TPOOL = B * 13 * 13


def _avg_mat(out_size, in_size):
    M = np.zeros((out_size, in_size), dtype=np.float32)
    for i in range(out_size):
        s = (i * in_size) // out_size
        e = int(np.ceil((i + 1) * in_size / out_size))
        M[i, s:e] = 1.0 / (e - s)
    return M


def _cubic_w(t, a=-0.75):
    t = abs(float(t))
    if t <= 1.0:
        return (a + 2.0) * t ** 3 - (a + 3.0) * t ** 2 + 1.0
    elif t < 2.0:
        return a * t ** 3 - 5.0 * a * t ** 2 + 8.0 * a * t - 4.0 * a
    return 0.0


def _bicubic_mat(out_size, in_size):
    M = np.zeros((out_size, in_size), dtype=np.float32)
    scale = in_size / out_size
    for i in range(out_size):
        src = (i + 0.5) * scale - 0.5
        i0 = int(np.floor(src))
        t = src - i0
        for k in range(-1, 3):
            w = _cubic_w(t - k)
            j = min(max(i0 + k, 0), in_size - 1)
            M[i, j] += w
    return M


_P1_MATS, _P2_MATS, _U1_MATS, _U2_MATS = [], [], [], []
for _p in MS[:-1]:
    _A = _avg_mat(_p, H)
    _P1_MATS.append(np.kron(_A, np.eye(W, dtype=np.float32)))
    _P2_MATS.append(np.kron(np.eye(_p, dtype=np.float32), _A))
    _U = _bicubic_mat(H, _p)
    _U1_MATS.append(np.kron(_U, np.eye(_p, dtype=np.float32)))
    _U2_MATS.append(np.kron(np.eye(H, dtype=np.float32), _U))


def _dot(a, b, dims):
    return lax.dot_general(a, b, (dims, ((), ())),
                           preferred_element_type=jnp.float32)


def _vq_kernel(zT_ref, emb_ref, *refs):
    NL = NLEV - 1
    p1_refs = refs[0:NL]
    p2_refs = refs[NL:2 * NL]
    u1_refs = refs[2 * NL:3 * NL]
    u2_refs = refs[3 * NL:4 * NL]
    iota_ref = refs[4 * NL]
    out_ref = refs[4 * NL + 1]
    z_rest, z_dec, zd_s, lut_s, idx_s, wn_s, sem = refs[4 * NL + 2:]

    z_rest[...] = zT_ref[...]
    z_dec[...] = jnp.zeros((TMAX, C), jnp.float32)
    ones_row = jnp.ones((1, C), jnp.float32)
    ones_col = jnp.ones((C, 1), jnp.float32)

    for kt in range(K // KT):
        embt = emb_ref[kt * KT:(kt + 1) * KT, :]
        wn_s[kt:kt + 1, :] = _dot(ones_row, embt * embt, (((1,), (1,))))

    for lev, p in enumerate(MS):
        psq = p * p
        T = B * psq
        if lev < NLEV - 1:
            P1v = p1_refs[lev][...]
            P2v = p2_refs[lev][...]
            zrv = z_rest[...]
            for b in range(B):
                t1 = _dot(P1v, zrv[b * HW:(b + 1) * HW, :], (((1,), (0,))))
                zd_s[b * psq:(b + 1) * psq, :] = _dot(P2v, t1, (((1,), (0,))))
            zdv = zd_s[0:T, :]
        else:
            zdv = z_rest[...]

        zn = _dot(zdv * zdv, ones_col, (((1,), (0,))))
        zdv2 = zdv + zdv

        def ktile(kt, carry):
            mval, midx = carry
            embt = emb_ref[pl.ds(kt * KT, KT), :]
            wn_t = wn_s[pl.ds(kt, 1), :]
            s2 = _dot(zdv2, embt, (((1,), (1,))))
            dq = (zn + wn_t) - s2
            tmin = jnp.min(dq, axis=1, keepdims=True)
            iota = iota_ref[...] + (kt * KT).astype(jnp.float32)
            tidx = jnp.min(jnp.where(dq == tmin, iota, jnp.float32(K)),
                           axis=1, keepdims=True)
            better = tmin < mval
            return (jnp.where(better, tmin, mval),
                    jnp.where(better, tidx, midx))

        mval, midx = lax.fori_loop(
            0, K // KT, ktile,
            (jnp.full((T, 1), jnp.inf, jnp.float32),
             jnp.full((T, 1), K, jnp.float32)), unroll=2)
        idx_s[0:T, :] = midx.astype(jnp.int32)

        def gbody(i, carry):
            k = idx_s[i, 0]
            lut_s[pl.ds(i, 1), :] = emb_ref[pl.ds(k, 1), :]
            return carry
        lax.fori_loop(0, T, gbody, 0, unroll=32)

        if lev < NLEV - 1:
            U1v = u1_refs[lev][...]
            U2v = u2_refs[lev][...]
            ups = []
            for b in range(B):
                t2 = _dot(U1v, lut_s[b * psq:(b + 1) * psq, :],
                          (((1,), (0,))))
                ups.append(_dot(U2v, t2, (((1,), (0,)))))
            up = jnp.concatenate(ups, axis=0)
        else:
            up = lut_s[...]
        z_dec[...] = z_dec[...] + up
        if lev < NLEV - 1:
            z_rest[...] = z_rest[...] - up

        cp = pltpu.make_async_copy(z_dec, out_ref.at[lev], sem)
        cp.start()
        cp.wait()


def kernel(z_enc, emb_weight):
    zT = jnp.transpose(z_enc, (0, 2, 3, 1)).reshape(TMAX, C)
    mat_consts = [jnp.asarray(m) for mats in
                  (_P1_MATS, _P2_MATS, _U1_MATS, _U2_MATS) for m in mats]
    mat_consts.append(jnp.arange(KT, dtype=jnp.float32).reshape(1, KT))

    out = pl.pallas_call(
        _vq_kernel,
        in_specs=[pl.BlockSpec(memory_space=pltpu.VMEM)] * (3 + 4 * (NLEV - 1)),
        out_specs=pl.BlockSpec(memory_space=pl.ANY),
        out_shape=jax.ShapeDtypeStruct((NLEV, TMAX, C), jnp.float32),
        scratch_shapes=[
            pltpu.VMEM((TMAX, C), jnp.float32),
            pltpu.VMEM((TMAX, C), jnp.float32),
            pltpu.VMEM((TPOOL, C), jnp.float32),
            pltpu.VMEM((TMAX, C), jnp.float32),
            pltpu.VMEM((TMAX, 1), jnp.int32),
            pltpu.VMEM((K // KT, KT), jnp.float32),
            pltpu.SemaphoreType.DMA,
        ],
    )(zT, emb_weight, *mat_consts)

    return out.reshape(NLEV, B, H, W, C).transpose(0, 1, 4, 2, 3)


# speedup vs baseline: 1.2786x; 1.2786x over previous
import numpy as np
import jax
import jax.numpy as jnp
from jax import lax
from jax.experimental import pallas as pl
from jax.experimental.pallas import tpu as pltpu

B, C, H, W = 16, 256, 16, 16
K = 8192
MS = [1, 2, 3, 4, 5, 6, 8, 10, 13, 16]
NLEV = len(MS)
KT = 512
HW = H * W
TMAX = B * HW
TPOOL = B * 13 * 13


def _avg_mat(out_size, in_size):
    M = np.zeros((out_size, in_size), dtype=np.float32)
    for i in range(out_size):
        s = (i * in_size) // out_size
        e = int(np.ceil((i + 1) * in_size / out_size))
        M[i, s:e] = 1.0 / (e - s)
    return M


def _cubic_w(t, a=-0.75):
    t = abs(float(t))
    if t <= 1.0:
        return (a + 2.0) * t ** 3 - (a + 3.0) * t ** 2 + 1.0
    elif t < 2.0:
        return a * t ** 3 - 5.0 * a * t ** 2 + 8.0 * a * t - 4.0 * a
    return 0.0


def _bicubic_mat(out_size, in_size):
    M = np.zeros((out_size, in_size), dtype=np.float32)
    scale = in_size / out_size
    for i in range(out_size):
        src = (i + 0.5) * scale - 0.5
        i0 = int(np.floor(src))
        t = src - i0
        for k in range(-1, 3):
            w = _cubic_w(t - k)
            j = min(max(i0 + k, 0), in_size - 1)
            M[i, j] += w
    return M


_P1_MATS, _P2_MATS, _U1_MATS, _U2_MATS = [], [], [], []
for _p in MS[:-1]:
    _A = _avg_mat(_p, H)
    _P1_MATS.append(np.kron(_A, np.eye(W, dtype=np.float32)))
    _P2_MATS.append(np.kron(np.eye(_p, dtype=np.float32), _A))
    _U = _bicubic_mat(H, _p)
    _U1_MATS.append(np.kron(_U, np.eye(_p, dtype=np.float32)))
    _U2_MATS.append(np.kron(np.eye(H, dtype=np.float32), _U))


def _dot(a, b, dims):
    return lax.dot_general(a, b, (dims, ((), ())),
                           preferred_element_type=jnp.float32)


def _vq_kernel(zT_ref, emb_ref, *refs):
    NL = NLEV - 1
    p1_refs = refs[0:NL]
    p2_refs = refs[NL:2 * NL]
    u1_refs = refs[2 * NL:3 * NL]
    u2_refs = refs[3 * NL:4 * NL]
    iota_ref = refs[4 * NL]
    out_ref = refs[4 * NL + 1]
    z_rest, z_dec, zd_s, lut_s, idx_s, wn_s, sem = refs[4 * NL + 2:]

    z_rest[...] = zT_ref[...]
    z_dec[...] = jnp.zeros((TMAX, C), jnp.float32)
    ones_row = jnp.ones((1, C), jnp.float32)
    ones_col = jnp.ones((C, 1), jnp.float32)

    for kt in range(K // KT):
        embt = emb_ref[kt * KT:(kt + 1) * KT, :]
        wn_s[kt:kt + 1, :] = _dot(ones_row, embt * embt, (((1,), (1,))))

    for lev, p in enumerate(MS):
        psq = p * p
        T = B * psq
        if lev < NLEV - 1:
            P1v = p1_refs[lev][...]
            P2v = p2_refs[lev][...]
            zrv = z_rest[...]
            for b in range(B):
                t1 = _dot(P1v, zrv[b * HW:(b + 1) * HW, :], (((1,), (0,))))
                zd_s[b * psq:(b + 1) * psq, :] = _dot(P2v, t1, (((1,), (0,))))
            zdv = zd_s[0:T, :]
        else:
            zdv = z_rest[...]

        zn = _dot(zdv * zdv, ones_col, (((1,), (0,))))
        zdv2 = zdv + zdv

        def ktile(kt, carry):
            mval, midx = carry
            embt = emb_ref[pl.ds(kt * KT, KT), :]
            wn_t = wn_s[pl.ds(kt, 1), :]
            s2 = _dot(zdv2, embt, (((1,), (1,))))
            dq = (zn + wn_t) - s2
            tmin = jnp.min(dq, axis=1, keepdims=True)
            iota = iota_ref[...] + (kt * KT).astype(jnp.float32)
            tidx = jnp.min(jnp.where(dq == tmin, iota, jnp.float32(K)),
                           axis=1, keepdims=True)
            better = tmin < mval
            return (jnp.where(better, tmin, mval),
                    jnp.where(better, tidx, midx))

        mval, midx = lax.fori_loop(
            0, K // KT, ktile,
            (jnp.full((T, 1), jnp.inf, jnp.float32),
             jnp.full((T, 1), K, jnp.float32)), unroll=2)
        idx_s[0:T, :] = midx.astype(jnp.int32)

        def gbody(i, carry):
            k = idx_s[i, 0]
            lut_s[pl.ds(i, 1), :] = emb_ref[pl.ds(k, 1), :]
            return carry
        lax.fori_loop(0, T, gbody, 0, unroll=32)

        if lev < NLEV - 1:
            U1v = u1_refs[lev][...]
            U2v = u2_refs[lev][...]
            ups = []
            for b in range(B):
                t2 = _dot(U1v, lut_s[b * psq:(b + 1) * psq, :],
                          (((1,), (0,))))
                ups.append(_dot(U2v, t2, (((1,), (0,)))))
            up = jnp.concatenate(ups, axis=0)
        else:
            up = lut_s[...]
        z_dec[...] = z_dec[...] + up
        if lev < NLEV - 1:
            z_rest[...] = z_rest[...] - up

        cp = pltpu.make_async_copy(z_dec, out_ref.at[lev], sem)
        cp.start()
        cp.wait()


def kernel(z_enc, emb_weight):
    zT = jnp.transpose(z_enc, (0, 2, 3, 1)).reshape(TMAX, C)
    mat_consts = [jnp.asarray(m) for mats in
                  (_P1_MATS, _P2_MATS, _U1_MATS, _U2_MATS) for m in mats]
    mat_consts.append(jnp.arange(KT, dtype=jnp.float32).reshape(1, KT))

    out = pl.pallas_call(
        _vq_kernel,
        in_specs=[pl.BlockSpec(memory_space=pltpu.VMEM)] * (3 + 4 * (NLEV - 1)),
        out_specs=pl.BlockSpec(memory_space=pl.ANY),
        out_shape=jax.ShapeDtypeStruct((NLEV, TMAX, C), jnp.float32),
        scratch_shapes=[
            pltpu.VMEM((TMAX, C), jnp.float32),
            pltpu.VMEM((TMAX, C), jnp.float32),
            pltpu.VMEM((TPOOL, C), jnp.float32),
            pltpu.VMEM((TMAX, C), jnp.float32),
            pltpu.VMEM((TMAX, 1), jnp.int32),
            pltpu.VMEM((K // KT, KT), jnp.float32),
            pltpu.SemaphoreType.DMA,
        ],
    )(zT, emb_weight, *mat_consts)

    return out.reshape(NLEV, B, H, W, C).transpose(0, 1, 4, 2, 3)
